# baseline (speedup 1.0000x reference)
import functools

import jax
import jax.numpy as jnp
from jax import lax
from jax.experimental import pallas as pl
from jax.experimental.pallas import tpu as pltpu
from jax.experimental.pallas import tpu_sc as plsc

NUM_CORES = 2
NUM_SUBCORES = 16
NUM_WORKERS = NUM_CORES * NUM_SUBCORES


@functools.lru_cache(maxsize=None)
def _make_lookup(batch: int, vsz: int, dsz: int, chunk: int):
    assert batch % NUM_WORKERS == 0
    b_per_w = batch // NUM_WORKERS
    assert b_per_w % chunk == 0
    n_chunks = b_per_w // chunk

    mesh = plsc.VectorSubcoreMesh(core_axis_name="c", subcore_axis_name="s")

    @functools.partial(
        pl.kernel,
        mesh=mesh,
        out_type=jax.ShapeDtypeStruct((batch, dsz), jnp.float32),
        scratch_types=[
            pltpu.VMEM((chunk,), jnp.int32),
            pltpu.VMEM((chunk, dsz), jnp.float32),
            pltpu.SemaphoreType.DMA,
        ],
        compiler_params=pltpu.CompilerParams(use_tc_tiling_on_sc=False),
    )
    def lookup(idx_hbm, table_hbm, out_hbm, idx_v, rows_v, sem):
        wid = lax.axis_index("s") * NUM_CORES + lax.axis_index("c")
        base = wid * b_per_w

        @pl.loop(0, n_chunks)
        def _chunk_body(i):
            off = base + i * chunk
            pltpu.sync_copy(idx_hbm.at[pl.ds(off, chunk)], idx_v)
            pltpu.async_copy(table_hbm.at[idx_v], rows_v, sem).wait()
            pltpu.sync_copy(rows_v, out_hbm.at[pl.ds(off, chunk)])

    return lookup


def kernel(x, table):
    bsz, hist = x.shape
    vsz, dsz = table.shape
    flat = x.reshape(bsz * hist)
    lookup = _make_lookup(bsz * hist, vsz, dsz, 512)
    out = lookup(flat, table)
    return out.reshape(bsz, hist, dsz)


# speedup vs baseline: 1.0393x; 1.0393x over previous
import functools

import jax
import jax.numpy as jnp
from jax import lax
from jax.experimental import pallas as pl
from jax.experimental.pallas import tpu as pltpu
from jax.experimental.pallas import tpu_sc as plsc

NUM_CORES = 2
NUM_SUBCORES = 16
NUM_WORKERS = NUM_CORES * NUM_SUBCORES
NBUF = 4


@functools.lru_cache(maxsize=None)
def _make_lookup(batch: int, vsz: int, dsz: int, chunk: int):
    assert batch % NUM_WORKERS == 0
    b_per_w = batch // NUM_WORKERS
    assert b_per_w % (chunk * NBUF) == 0
    n_outer = b_per_w // (chunk * NBUF)

    mesh = plsc.VectorSubcoreMesh(core_axis_name="c", subcore_axis_name="s")

    @functools.partial(
        pl.kernel,
        mesh=mesh,
        out_type=jax.ShapeDtypeStruct((batch, dsz), jnp.float32),
        scratch_types=(
            [pltpu.VMEM((b_per_w,), jnp.int32)]
            + [pltpu.VMEM((chunk, dsz), jnp.float32) for _ in range(NBUF)]
            + [pltpu.SemaphoreType.DMA for _ in range(2 * NBUF)]
        ),
        compiler_params=pltpu.CompilerParams(use_tc_tiling_on_sc=False),
    )
    def lookup(idx_hbm, table_hbm, out_hbm, idx_v, *bufs_and_sems):
        rows = bufs_and_sems[:NBUF]
        gsem = bufs_and_sems[NBUF:2 * NBUF]
        ssem = bufs_and_sems[2 * NBUF:]
        wid = lax.axis_index("s") * NUM_CORES + lax.axis_index("c")
        base = wid * b_per_w

        pltpu.sync_copy(idx_hbm.at[pl.ds(base, b_per_w)], idx_v)

        def start_gather(i, b):
            pltpu.async_copy(
                table_hbm.at[idx_v.at[pl.ds(i * chunk, chunk)]],
                rows[b], gsem[b])

        def start_store(i, b):
            pltpu.async_copy(rows[b], out_hbm.at[pl.ds(base + i * chunk, chunk)],
                             ssem[b])

        def wait_gather(b):
            pltpu.make_async_copy(
                out_hbm.at[pl.ds(0, chunk)], rows[b], gsem[b]).wait()

        def wait_store(b):
            pltpu.make_async_copy(
                rows[b], out_hbm.at[pl.ds(0, chunk)], ssem[b]).wait()

        for b in range(NBUF):
            start_gather(b, b)

        @pl.loop(0, n_outer - 1)
        def _round(j):
            i0 = j * NBUF
            for b in range(NBUF):
                wait_gather(b)
                start_store(i0 + b, b)
            for b in range(NBUF):
                wait_store(b)
                start_gather(i0 + NBUF + b, b)

        i0 = (n_outer - 1) * NBUF
        for b in range(NBUF):
            wait_gather(b)
            start_store(i0 + b, b)
        for b in range(NBUF):
            wait_store(b)

    return lookup


def kernel(x, table):
    bsz, hist = x.shape
    vsz, dsz = table.shape
    flat = x.reshape(bsz * hist)
    lookup = _make_lookup(bsz * hist, vsz, dsz, 256)
    out = lookup(flat, table)
    return out.reshape(bsz, hist, dsz)
